# baseline (speedup 1.0000x reference)
import functools
import math

import jax
import jax.numpy as jnp
from jax import lax
from jax.experimental import pallas as pl
from jax.experimental.pallas import tpu as pltpu
from jax.experimental.pallas import tpu_sc as plsc

_FEATURE_DIM = 128
_ATTN_DIM = 32
_TOP_K = 4
_DECAY_RATE = 0.95
_TAU = 1.0
_T = 128
_P = 4096
_BP = 128
_BC = 512

_NUM_CORES = 2
_NUM_SUBCORES = 16
_NUM_WORKERS = _NUM_CORES * _NUM_SUBCORES
_ROWS_PER_WORKER = _TOP_K * _P // _NUM_WORKERS
_GATHER_CHUNK = 128


def _score_body(q_ref, h_ref, wq_ref, wk_ref, idx_ref, w_ref):
    T, bp, A, D, P = _T, _BP, _ATTN_DIM, _FEATURE_DIM, _P
    bf = jnp.bfloat16

    q = q_ref[...].astype(bf)
    Q = jnp.dot(q, wq_ref[...].astype(bf).T, preferred_element_type=jnp.float32)
    Q = Q.astype(bf).astype(jnp.float32)

    h = h_ref[...]
    hf = h.reshape(T * bp, D).astype(bf)
    K = jnp.dot(hf, wk_ref[...].astype(bf).T, preferred_element_type=jnp.float32)
    K3 = K.astype(bf).T.reshape(A, T, bp).astype(jnp.float32)
    scores = (K3 * Q.T[:, None, :]).sum(axis=0) * (1.0 / math.sqrt(A))
    st = scores.T

    iota_t = lax.broadcasted_iota(jnp.int32, (bp, T), 1)
    neg_inf = jnp.float32(-jnp.inf)

    sc = st
    vals = []
    idxs = []
    for _ in range(_TOP_K):
        m = jnp.max(sc, axis=1)
        cand = jnp.where(sc == m[:, None], iota_t, T)
        i = jnp.min(cand, axis=1)
        vals.append(m)
        idxs.append(i)
        sc = jnp.where(iota_t == i[:, None], neg_inf, sc)

    log_decay = math.log(_DECAY_RATE)
    zs = []
    for m, i in zip(vals, idxs):
        delta = (T - i).astype(jnp.float32)
        bias = jnp.log(jnp.exp(delta * log_decay) + 1e-10)
        zs.append((m + bias) * (1.0 / _TAU))
    zm = zs[0]
    for z in zs[1:]:
        zm = jnp.maximum(zm, z)
    es = [jnp.exp(z - zm) for z in zs]
    denom = es[0]
    for e in es[1:]:
        denom = denom + e
    ws = [e / denom for e in es]

    p_global = pl.program_id(0) * bp + lax.broadcasted_iota(jnp.int32, (bp,), 0)
    rows = [i * P + p_global for i in idxs]
    idx_ref[...] = jnp.concatenate([r[None, :] for r in rows], axis=0)
    w_ref[...] = jnp.concatenate([w[None, :] for w in ws], axis=0)


def _gather_body(h_ref, idx_ref, out_ref, i0, i1, i2, i3, rows_v, sem):
    wid = lax.axis_index("s") * _NUM_CORES + lax.axis_index("c")
    base = wid * _ROWS_PER_WORKER
    chunks = (i0, i1, i2, i3)
    for j, iv in enumerate(chunks):
        pltpu.sync_copy(idx_ref.at[pl.ds(base + j * _GATHER_CHUNK, _GATHER_CHUNK)], iv)
    copies = []
    for j, iv in enumerate(chunks):
        copies.append(pltpu.async_copy(
            h_ref.at[iv], rows_v.at[pl.ds(j * _GATHER_CHUNK, _GATHER_CHUNK)], sem))
    for c in copies:
        c.wait()
    pltpu.sync_copy(rows_v, out_ref.at[pl.ds(base, _ROWS_PER_WORKER)])


def _combine_body(rows_ref, w_ref, wv_ref, wo_ref, o_ref):
    bc, A, D = _BC, _ATTN_DIM, _FEATURE_DIM
    bf = jnp.bfloat16
    w4 = w_ref[...]
    wvT = wv_ref[...].astype(bf).T
    acc = jnp.zeros((bc, A), dtype=jnp.float32)
    for k in range(_TOP_K):
        rk = rows_ref[k]
        Vk = jnp.dot(rk.astype(bf), wvT, preferred_element_type=jnp.float32)
        Vk = Vk.astype(bf).astype(jnp.float32)
        acc = acc + Vk * w4[k][:, None]
    o_ref[...] = jnp.dot(acc.astype(bf), wo_ref[...].astype(bf).T,
                         preferred_element_type=jnp.float32)


def kernel(query_features, history_buffer, W_q, W_k, W_v, W_o):
    H, W, D = query_features.shape
    T = history_buffer.shape[0]
    P = H * W
    A = _ATTN_DIM
    q2 = query_features.reshape(P, D)
    h3 = history_buffer.reshape(T, P, D)
    h2 = history_buffer.reshape(T * P, D)

    idx, wts = pl.pallas_call(
        _score_body,
        grid=(P // _BP,),
        in_specs=[
            pl.BlockSpec((_BP, D), lambda i: (i, 0)),
            pl.BlockSpec((T, _BP, D), lambda i: (0, i, 0)),
            pl.BlockSpec((A, D), lambda i: (0, 0)),
            pl.BlockSpec((A, D), lambda i: (0, 0)),
        ],
        out_specs=[
            pl.BlockSpec((_TOP_K, _BP), lambda i: (0, i)),
            pl.BlockSpec((_TOP_K, _BP), lambda i: (0, i)),
        ],
        out_shape=[
            jax.ShapeDtypeStruct((_TOP_K, P), jnp.int32),
            jax.ShapeDtypeStruct((_TOP_K, P), jnp.float32),
        ],
    )(q2, h3, W_q, W_k)

    idx_flat = idx.reshape(_TOP_K * P)

    mesh = plsc.VectorSubcoreMesh(core_axis_name="c", subcore_axis_name="s")
    gather = pl.kernel(
        _gather_body,
        out_type=jax.ShapeDtypeStruct((_TOP_K * P, D), jnp.float32),
        mesh=mesh,
        scratch_types=[
            pltpu.VMEM((_GATHER_CHUNK,), jnp.int32),
            pltpu.VMEM((_GATHER_CHUNK,), jnp.int32),
            pltpu.VMEM((_GATHER_CHUNK,), jnp.int32),
            pltpu.VMEM((_GATHER_CHUNK,), jnp.int32),
            pltpu.VMEM((_ROWS_PER_WORKER, D), jnp.float32),
            pltpu.SemaphoreType.DMA,
        ],
    )
    rows = gather(h2, idx_flat)
    rows4 = rows.reshape(_TOP_K, P, D)

    out = pl.pallas_call(
        _combine_body,
        grid=(P // _BC,),
        in_specs=[
            pl.BlockSpec((_TOP_K, _BC, D), lambda i: (0, i, 0)),
            pl.BlockSpec((_TOP_K, _BC), lambda i: (0, i)),
            pl.BlockSpec((A, D), lambda i: (0, 0)),
            pl.BlockSpec((D, A), lambda i: (0, 0)),
        ],
        out_specs=pl.BlockSpec((_BC, D), lambda i: (i, 0)),
        out_shape=jax.ShapeDtypeStruct((P, D), jnp.float32),
    )(rows4, wts, W_v, W_o)
    return out.reshape(H, W, D)


# speedup vs baseline: 1.0909x; 1.0909x over previous
import math

import jax
import jax.numpy as jnp
from jax import lax
from jax.experimental import pallas as pl
from jax.experimental.pallas import tpu as pltpu
from jax.experimental.pallas import tpu_sc as plsc

_FEATURE_DIM = 128
_ATTN_DIM = 32
_TOP_K = 4
_DECAY_RATE = 0.95
_TAU = 1.0
_T = 128
_P = 4096
_BP = 256
_BC = 512

_NUM_CORES = 2
_NUM_SUBCORES = 16
_NUM_WORKERS = _NUM_CORES * _NUM_SUBCORES
_ROWS_PER_WORKER = _TOP_K * _P // _NUM_WORKERS
_GATHER_CHUNK = 128


def _score_body(q_ref, h_ref, wq_ref, wk_ref, idx_ref, w_ref):
    T, bp, A, D, P = _T, _BP, _ATTN_DIM, _FEATURE_DIM, _P
    bf = jnp.bfloat16

    q = q_ref[...].astype(bf)
    Q = jnp.dot(q, wq_ref[...].astype(bf).T, preferred_element_type=jnp.float32)
    Q = Q.astype(bf).astype(jnp.float32)

    h = h_ref[...]
    hf = h.reshape(T * bp, D).astype(bf)
    K = jnp.dot(hf, wk_ref[...].astype(bf).T, preferred_element_type=jnp.float32)
    K3 = K.astype(bf).T.reshape(A, T, bp).astype(jnp.float32)
    scores = (K3 * Q.T[:, None, :]).sum(axis=0) * (1.0 / math.sqrt(A))
    st = scores.T

    iota_t = lax.broadcasted_iota(jnp.int32, (bp, T), 1)
    neg_inf = jnp.float32(-jnp.inf)

    sc = st
    vals = []
    idxs = []
    for _ in range(_TOP_K):
        m = jnp.max(sc, axis=1)
        cand = jnp.where(sc == m[:, None], iota_t, T)
        i = jnp.min(cand, axis=1)
        vals.append(m)
        idxs.append(i)
        sc = jnp.where(iota_t == i[:, None], neg_inf, sc)

    log_decay = math.log(_DECAY_RATE)
    zs = []
    for m, i in zip(vals, idxs):
        delta = (T - i).astype(jnp.float32)
        bias = jnp.log(jnp.exp(delta * log_decay) + 1e-10)
        zs.append((m + bias) * (1.0 / _TAU))
    zm = zs[0]
    for z in zs[1:]:
        zm = jnp.maximum(zm, z)
    es = [jnp.exp(z - zm) for z in zs]
    denom = es[0]
    for e in es[1:]:
        denom = denom + e
    ws = [e / denom for e in es]

    p_global = pl.program_id(0) * bp + lax.broadcasted_iota(jnp.int32, (bp,), 0)
    rows = [i * P + p_global for i in idxs]
    idx_ref[...] = jnp.concatenate([r[None, :] for r in rows], axis=0)
    w_ref[...] = jnp.concatenate([w[None, :] for w in ws], axis=0)


def _gather_body(h_ref, idx_ref, out_ref, i0, i1, i2, i3, rows_v, sem):
    wid = lax.axis_index("s") * _NUM_CORES + lax.axis_index("c")
    base = wid * _ROWS_PER_WORKER
    chunks = (i0, i1, i2, i3)
    for j, iv in enumerate(chunks):
        pltpu.sync_copy(idx_ref.at[pl.ds(base + j * _GATHER_CHUNK, _GATHER_CHUNK)], iv)
    copies = []
    for j, iv in enumerate(chunks):
        copies.append(pltpu.async_copy(
            h_ref.at[iv], rows_v.at[pl.ds(j * _GATHER_CHUNK, _GATHER_CHUNK)], sem))
    for c in copies:
        c.wait()
    pltpu.sync_copy(rows_v, out_ref.at[pl.ds(base, _ROWS_PER_WORKER)])


def _combine_body(rows_ref, w_ref, wv_ref, wo_ref, o_ref):
    bc, A, D = _BC, _ATTN_DIM, _FEATURE_DIM
    bf = jnp.bfloat16
    w4 = w_ref[...]
    wvT = wv_ref[...].astype(bf).T
    acc = jnp.zeros((bc, A), dtype=jnp.float32)
    for k in range(_TOP_K):
        rk = rows_ref[k]
        Vk = jnp.dot(rk.astype(bf), wvT, preferred_element_type=jnp.float32)
        Vk = Vk.astype(bf).astype(jnp.float32)
        acc = acc + Vk * w4[k][:, None]
    o_ref[...] = jnp.dot(acc.astype(bf), wo_ref[...].astype(bf).T,
                         preferred_element_type=jnp.float32)


def kernel(query_features, history_buffer, W_q, W_k, W_v, W_o):
    H, W, D = query_features.shape
    T = history_buffer.shape[0]
    P = H * W
    A = _ATTN_DIM
    q2 = query_features.reshape(P, D)
    h3 = history_buffer.reshape(T, P, D)
    h2 = history_buffer.reshape(T * P, D)

    idx, wts = pl.pallas_call(
        _score_body,
        grid=(P // _BP,),
        in_specs=[
            pl.BlockSpec((_BP, D), lambda i: (i, 0)),
            pl.BlockSpec((T, _BP, D), lambda i: (0, i, 0)),
            pl.BlockSpec((A, D), lambda i: (0, 0)),
            pl.BlockSpec((A, D), lambda i: (0, 0)),
        ],
        out_specs=[
            pl.BlockSpec((_TOP_K, _BP), lambda i: (0, i)),
            pl.BlockSpec((_TOP_K, _BP), lambda i: (0, i)),
        ],
        out_shape=[
            jax.ShapeDtypeStruct((_TOP_K, P), jnp.int32),
            jax.ShapeDtypeStruct((_TOP_K, P), jnp.float32),
        ],
    )(q2, h3, W_q, W_k)

    idx_flat = idx.reshape(_TOP_K * P)

    mesh = plsc.VectorSubcoreMesh(core_axis_name="c", subcore_axis_name="s")
    gather = pl.kernel(
        _gather_body,
        out_type=jax.ShapeDtypeStruct((_TOP_K * P, D), jnp.float32),
        mesh=mesh,
        scratch_types=[
            pltpu.VMEM((_GATHER_CHUNK,), jnp.int32),
            pltpu.VMEM((_GATHER_CHUNK,), jnp.int32),
            pltpu.VMEM((_GATHER_CHUNK,), jnp.int32),
            pltpu.VMEM((_GATHER_CHUNK,), jnp.int32),
            pltpu.VMEM((_ROWS_PER_WORKER, D), jnp.float32),
            pltpu.SemaphoreType.DMA,
        ],
    )
    rows = gather(h2, idx_flat)
    rows4 = rows.reshape(_TOP_K, P, D)

    out = pl.pallas_call(
        _combine_body,
        grid=(P // _BC,),
        in_specs=[
            pl.BlockSpec((_TOP_K, _BC, D), lambda i: (0, i, 0)),
            pl.BlockSpec((_TOP_K, _BC), lambda i: (0, i)),
            pl.BlockSpec((A, D), lambda i: (0, 0)),
            pl.BlockSpec((D, A), lambda i: (0, 0)),
        ],
        out_specs=pl.BlockSpec((_BC, D), lambda i: (i, 0)),
        out_shape=jax.ShapeDtypeStruct((P, D), jnp.float32),
    )(rows4, wts, W_v, W_o)
    return out.reshape(H, W, D)


# speedup vs baseline: 1.1086x; 1.0163x over previous
import math

import jax
import jax.numpy as jnp
from jax import lax
from jax.experimental import pallas as pl
from jax.experimental.pallas import tpu as pltpu
from jax.experimental.pallas import tpu_sc as plsc

_FEATURE_DIM = 128
_ATTN_DIM = 32
_TOP_K = 4
_DECAY_RATE = 0.95
_TAU = 1.0
_T = 128
_P = 4096
_BP = 256
_BC = 1024

_NUM_CORES = 2
_NUM_SUBCORES = 16
_NUM_WORKERS = _NUM_CORES * _NUM_SUBCORES
_ROWS_PER_WORKER = _TOP_K * _P // _NUM_WORKERS
_GATHER_CHUNK = 128


def _score_body(q_ref, h_ref, wq_ref, wk_ref, idx_ref, w_ref):
    T, bp, A, D, P = _T, _BP, _ATTN_DIM, _FEATURE_DIM, _P
    bf = jnp.bfloat16

    q = q_ref[...].astype(bf)
    Q = jnp.dot(q, wq_ref[...].astype(bf).T, preferred_element_type=jnp.float32)
    Q = Q.astype(bf).astype(jnp.float32)

    h = h_ref[...]
    hf = h.reshape(T * bp, D).astype(bf)
    K = jnp.dot(hf, wk_ref[...].astype(bf).T, preferred_element_type=jnp.float32)
    K3 = K.astype(bf).T.reshape(A, T, bp).astype(jnp.float32)
    scores = (K3 * Q.T[:, None, :]).sum(axis=0) * (1.0 / math.sqrt(A))
    st = scores.T

    iota_t = lax.broadcasted_iota(jnp.int32, (bp, T), 1)
    neg_inf = jnp.float32(-jnp.inf)

    sc = st
    vals = []
    idxs = []
    for _ in range(_TOP_K):
        m = jnp.max(sc, axis=1)
        cand = jnp.where(sc == m[:, None], iota_t, T)
        i = jnp.min(cand, axis=1)
        vals.append(m)
        idxs.append(i)
        sc = jnp.where(iota_t == i[:, None], neg_inf, sc)

    log_decay = math.log(_DECAY_RATE)
    zs = []
    for m, i in zip(vals, idxs):
        delta = (T - i).astype(jnp.float32)
        bias = jnp.log(jnp.exp(delta * log_decay) + 1e-10)
        zs.append((m + bias) * (1.0 / _TAU))
    zm = zs[0]
    for z in zs[1:]:
        zm = jnp.maximum(zm, z)
    es = [jnp.exp(z - zm) for z in zs]
    denom = es[0]
    for e in es[1:]:
        denom = denom + e
    ws = [e / denom for e in es]

    p_global = pl.program_id(0) * bp + lax.broadcasted_iota(jnp.int32, (bp,), 0)
    rows = [i * P + p_global for i in idxs]
    idx_ref[...] = jnp.concatenate([r[None, :] for r in rows], axis=0)
    w_ref[...] = jnp.concatenate([w[None, :] for w in ws], axis=0)


def _gather_body(h_ref, idx_ref, out_ref, i0, i1, i2, i3, rows_v, sem):
    wid = lax.axis_index("s") * _NUM_CORES + lax.axis_index("c")
    base = wid * _ROWS_PER_WORKER
    chunks = (i0, i1, i2, i3)
    for j, iv in enumerate(chunks):
        pltpu.sync_copy(idx_ref.at[pl.ds(base + j * _GATHER_CHUNK, _GATHER_CHUNK)], iv)
    copies = []
    for j, iv in enumerate(chunks):
        copies.append(pltpu.async_copy(
            h_ref.at[iv], rows_v.at[pl.ds(j * _GATHER_CHUNK, _GATHER_CHUNK)], sem))
    for c in copies:
        c.wait()
    pltpu.sync_copy(rows_v, out_ref.at[pl.ds(base, _ROWS_PER_WORKER)])


def _combine_body(rows_ref, w_ref, wv_ref, wo_ref, o_ref):
    bc, A, D = _BC, _ATTN_DIM, _FEATURE_DIM
    bf = jnp.bfloat16
    w4 = w_ref[...]
    wvT = wv_ref[...].astype(bf).T
    acc = jnp.zeros((bc, A), dtype=jnp.float32)
    for k in range(_TOP_K):
        rk = rows_ref[k]
        Vk = jnp.dot(rk.astype(bf), wvT, preferred_element_type=jnp.float32)
        Vk = Vk.astype(bf).astype(jnp.float32)
        acc = acc + Vk * w4[k][:, None]
    o_ref[...] = jnp.dot(acc.astype(bf), wo_ref[...].astype(bf).T,
                         preferred_element_type=jnp.float32)


def kernel(query_features, history_buffer, W_q, W_k, W_v, W_o):
    H, W, D = query_features.shape
    T = history_buffer.shape[0]
    P = H * W
    A = _ATTN_DIM
    q2 = query_features.reshape(P, D)
    h3 = history_buffer.reshape(T, P, D)
    h2 = history_buffer.reshape(T * P, D)

    idx, wts = pl.pallas_call(
        _score_body,
        grid=(P // _BP,),
        in_specs=[
            pl.BlockSpec((_BP, D), lambda i: (i, 0)),
            pl.BlockSpec((T, _BP, D), lambda i: (0, i, 0)),
            pl.BlockSpec((A, D), lambda i: (0, 0)),
            pl.BlockSpec((A, D), lambda i: (0, 0)),
        ],
        out_specs=[
            pl.BlockSpec((_TOP_K, _BP), lambda i: (0, i)),
            pl.BlockSpec((_TOP_K, _BP), lambda i: (0, i)),
        ],
        out_shape=[
            jax.ShapeDtypeStruct((_TOP_K, P), jnp.int32),
            jax.ShapeDtypeStruct((_TOP_K, P), jnp.float32),
        ],
    )(q2, h3, W_q, W_k)

    idx_flat = idx.reshape(_TOP_K * P)

    mesh = plsc.VectorSubcoreMesh(core_axis_name="c", subcore_axis_name="s")
    gather = pl.kernel(
        _gather_body,
        out_type=jax.ShapeDtypeStruct((_TOP_K * P, D), jnp.float32),
        mesh=mesh,
        scratch_types=[
            pltpu.VMEM((_GATHER_CHUNK,), jnp.int32),
            pltpu.VMEM((_GATHER_CHUNK,), jnp.int32),
            pltpu.VMEM((_GATHER_CHUNK,), jnp.int32),
            pltpu.VMEM((_GATHER_CHUNK,), jnp.int32),
            pltpu.VMEM((_ROWS_PER_WORKER, D), jnp.float32),
            pltpu.SemaphoreType.DMA,
        ],
    )
    rows = gather(h2, idx_flat)
    rows4 = rows.reshape(_TOP_K, P, D)

    out = pl.pallas_call(
        _combine_body,
        grid=(P // _BC,),
        in_specs=[
            pl.BlockSpec((_TOP_K, _BC, D), lambda i: (0, i, 0)),
            pl.BlockSpec((_TOP_K, _BC), lambda i: (0, i)),
            pl.BlockSpec((A, D), lambda i: (0, 0)),
            pl.BlockSpec((D, A), lambda i: (0, 0)),
        ],
        out_specs=pl.BlockSpec((_BC, D), lambda i: (i, 0)),
        out_shape=jax.ShapeDtypeStruct((P, D), jnp.float32),
    )(rows4, wts, W_v, W_o)
    return out.reshape(H, W, D)


# speedup vs baseline: 1.1175x; 1.0080x over previous
import math

import jax
import jax.numpy as jnp
from jax import lax
from jax.experimental import pallas as pl
from jax.experimental.pallas import tpu as pltpu
from jax.experimental.pallas import tpu_sc as plsc

_FEATURE_DIM = 128
_ATTN_DIM = 32
_TOP_K = 4
_DECAY_RATE = 0.95
_TAU = 1.0
_T = 128
_P = 4096
_BP = 256
_BC = 2048

_NUM_CORES = 2
_NUM_SUBCORES = 16
_NUM_WORKERS = _NUM_CORES * _NUM_SUBCORES
_ROWS_PER_WORKER = _TOP_K * _P // _NUM_WORKERS
_GATHER_CHUNK = 128


def _score_body(q_ref, h_ref, wq_ref, wk_ref, idx_ref, w_ref):
    T, bp, A, D, P = _T, _BP, _ATTN_DIM, _FEATURE_DIM, _P
    bf = jnp.bfloat16

    q = q_ref[...].astype(bf)
    Q = jnp.dot(q, wq_ref[...].astype(bf).T, preferred_element_type=jnp.float32)
    Q = Q.astype(bf).astype(jnp.float32)

    h = h_ref[...]
    hf = h.reshape(T * bp, D).astype(bf)
    K = jnp.dot(hf, wk_ref[...].astype(bf).T, preferred_element_type=jnp.float32)
    K3 = K.astype(bf).T.reshape(A, T, bp).astype(jnp.float32)
    scores = (K3 * Q.T[:, None, :]).sum(axis=0) * (1.0 / math.sqrt(A))
    st = scores.T

    iota_t = lax.broadcasted_iota(jnp.int32, (bp, T), 1)
    neg_inf = jnp.float32(-jnp.inf)

    sc = st
    vals = []
    idxs = []
    for _ in range(_TOP_K):
        m = jnp.max(sc, axis=1)
        cand = jnp.where(sc == m[:, None], iota_t, T)
        i = jnp.min(cand, axis=1)
        vals.append(m)
        idxs.append(i)
        sc = jnp.where(iota_t == i[:, None], neg_inf, sc)

    log_decay = math.log(_DECAY_RATE)
    zs = []
    for m, i in zip(vals, idxs):
        delta = (T - i).astype(jnp.float32)
        bias = jnp.log(jnp.exp(delta * log_decay) + 1e-10)
        zs.append((m + bias) * (1.0 / _TAU))
    zm = zs[0]
    for z in zs[1:]:
        zm = jnp.maximum(zm, z)
    es = [jnp.exp(z - zm) for z in zs]
    denom = es[0]
    for e in es[1:]:
        denom = denom + e
    ws = [e / denom for e in es]

    p_global = pl.program_id(0) * bp + lax.broadcasted_iota(jnp.int32, (bp,), 0)
    rows = [i * P + p_global for i in idxs]
    idx_ref[...] = jnp.concatenate([r[None, :] for r in rows], axis=0)
    w_ref[...] = jnp.concatenate([w[None, :] for w in ws], axis=0)


def _gather_body(h_ref, idx_ref, out_ref, i0, i1, i2, i3, rows_v, sem):
    wid = lax.axis_index("s") * _NUM_CORES + lax.axis_index("c")
    base = wid * _ROWS_PER_WORKER
    chunks = (i0, i1, i2, i3)
    for j, iv in enumerate(chunks):
        pltpu.sync_copy(idx_ref.at[pl.ds(base + j * _GATHER_CHUNK, _GATHER_CHUNK)], iv)
    copies = []
    for j, iv in enumerate(chunks):
        copies.append(pltpu.async_copy(
            h_ref.at[iv], rows_v.at[pl.ds(j * _GATHER_CHUNK, _GATHER_CHUNK)], sem))
    for c in copies:
        c.wait()
    pltpu.sync_copy(rows_v, out_ref.at[pl.ds(base, _ROWS_PER_WORKER)])


def _combine_body(rows_ref, w_ref, wv_ref, wo_ref, o_ref):
    bc, A, D = _BC, _ATTN_DIM, _FEATURE_DIM
    bf = jnp.bfloat16
    w4 = w_ref[...]
    wvT = wv_ref[...].astype(bf).T
    acc = jnp.zeros((bc, A), dtype=jnp.float32)
    for k in range(_TOP_K):
        rk = rows_ref[k]
        Vk = jnp.dot(rk.astype(bf), wvT, preferred_element_type=jnp.float32)
        Vk = Vk.astype(bf).astype(jnp.float32)
        acc = acc + Vk * w4[k][:, None]
    o_ref[...] = jnp.dot(acc.astype(bf), wo_ref[...].astype(bf).T,
                         preferred_element_type=jnp.float32)


def kernel(query_features, history_buffer, W_q, W_k, W_v, W_o):
    H, W, D = query_features.shape
    T = history_buffer.shape[0]
    P = H * W
    A = _ATTN_DIM
    q2 = query_features.reshape(P, D)
    h3 = history_buffer.reshape(T, P, D)
    h2 = history_buffer.reshape(T * P, D)

    idx, wts = pl.pallas_call(
        _score_body,
        grid=(P // _BP,),
        in_specs=[
            pl.BlockSpec((_BP, D), lambda i: (i, 0)),
            pl.BlockSpec((T, _BP, D), lambda i: (0, i, 0)),
            pl.BlockSpec((A, D), lambda i: (0, 0)),
            pl.BlockSpec((A, D), lambda i: (0, 0)),
        ],
        out_specs=[
            pl.BlockSpec((_TOP_K, _BP), lambda i: (0, i)),
            pl.BlockSpec((_TOP_K, _BP), lambda i: (0, i)),
        ],
        out_shape=[
            jax.ShapeDtypeStruct((_TOP_K, P), jnp.int32),
            jax.ShapeDtypeStruct((_TOP_K, P), jnp.float32),
        ],
    )(q2, h3, W_q, W_k)

    idx_flat = idx.reshape(_TOP_K * P)

    mesh = plsc.VectorSubcoreMesh(core_axis_name="c", subcore_axis_name="s")
    gather = pl.kernel(
        _gather_body,
        out_type=jax.ShapeDtypeStruct((_TOP_K * P, D), jnp.float32),
        mesh=mesh,
        scratch_types=[
            pltpu.VMEM((_GATHER_CHUNK,), jnp.int32),
            pltpu.VMEM((_GATHER_CHUNK,), jnp.int32),
            pltpu.VMEM((_GATHER_CHUNK,), jnp.int32),
            pltpu.VMEM((_GATHER_CHUNK,), jnp.int32),
            pltpu.VMEM((_ROWS_PER_WORKER, D), jnp.float32),
            pltpu.SemaphoreType.DMA,
        ],
    )
    rows = gather(h2, idx_flat)
    rows4 = rows.reshape(_TOP_K, P, D)

    out = pl.pallas_call(
        _combine_body,
        grid=(P // _BC,),
        in_specs=[
            pl.BlockSpec((_TOP_K, _BC, D), lambda i: (0, i, 0)),
            pl.BlockSpec((_TOP_K, _BC), lambda i: (0, i)),
            pl.BlockSpec((A, D), lambda i: (0, 0)),
            pl.BlockSpec((D, A), lambda i: (0, 0)),
        ],
        out_specs=pl.BlockSpec((_BC, D), lambda i: (i, 0)),
        out_shape=jax.ShapeDtypeStruct((P, D), jnp.float32),
    )(rows4, wts, W_v, W_o)
    return out.reshape(H, W, D)
